# baseline (speedup 1.0000x reference)
import functools

import jax
import jax.numpy as jnp
from jax import lax
from jax.experimental import pallas as pl
from jax.experimental.pallas import tpu as pltpu
from jax.experimental.pallas import tpu_sc as plsc

N = 10000
D = 128
E = 320000

NC = 2
NS = 16
NW = NC * NS

CHUNK = 128
CHUNKS_PER_TILE = 80
IDXB = 40
NPHASE = CHUNKS_PER_TILE // IDXB
EPAD = NW * CHUNKS_PER_TILE * CHUNK
ACC_ROWS = N + 16
ROWS_PER_TILE = 624
ROWS_TAIL = N - NS * ROWS_PER_TILE

def _sc_mesh():
    return plsc.VectorSubcoreMesh(core_axis_name="c", subcore_axis_name="s")


def _segsum_sc(h, zeros, src2d, dst2d):
    @functools.partial(
        pl.kernel,
        mesh=_sc_mesh(),
        out_type=jax.ShapeDtypeStruct((NC, N, D), jnp.float32),
        scratch_types=[
            pltpu.VMEM_SHARED((ACC_ROWS, D), jnp.float32),
            pltpu.VMEM((CHUNK,), jnp.int32),
            pltpu.VMEM((CHUNK,), jnp.int32),
            pltpu.VMEM((CHUNK,), jnp.int32),
            pltpu.VMEM((CHUNK,), jnp.int32),
            pltpu.VMEM((CHUNK, D), jnp.float32),
            pltpu.VMEM((CHUNK, D), jnp.float32),
            pltpu.SemaphoreType.DMA,
            pltpu.SemaphoreType.DMA,
            pltpu.SemaphoreType.DMA,
            pltpu.SemaphoreType.DMA,
        ],
    )
    def k(h_hbm, z_hbm, src_hbm, dst_hbm, out_hbm,
          acc, srcva, dstva, srcvb, dstvb, rows0, rows1,
          semi0, semi1, sem0, sem1):
        c = lax.axis_index("c")
        s = lax.axis_index("s")
        w = s * NC + c
        r0 = s * ROWS_PER_TILE
        cbase = w * (CHUNKS_PER_TILE * CHUNK)

        @pl.when(c == 0)
        def _():
            pltpu.sync_copy(h_hbm.at[pl.ds(r0, ROWS_PER_TILE)],
                            acc.at[pl.ds(r0, ROWS_PER_TILE)])

        @pl.when(c != 0)
        def _():
            pltpu.sync_copy(z_hbm.at[pl.ds(r0, ROWS_PER_TILE)],
                            acc.at[pl.ds(r0, ROWS_PER_TILE)])

        @pl.when((c == 0) & (s == NS - 1))
        def _():
            pltpu.sync_copy(h_hbm.at[pl.ds(NS * ROWS_PER_TILE, ROWS_TAIL)],
                            acc.at[pl.ds(NS * ROWS_PER_TILE, ROWS_TAIL)])

        @pl.when((c != 0) & (s == NS - 1))
        def _():
            pltpu.sync_copy(z_hbm.at[pl.ds(NS * ROWS_PER_TILE, ROWS_TAIL)],
                            acc.at[pl.ds(NS * ROWS_PER_TILE, ROWS_TAIL)])

        plsc.subcore_barrier()

        @pl.loop(0, CHUNKS_PER_TILE // 2)
        def _(jj):
            e0 = cbase + 2 * jj * CHUNK
            ia0 = pltpu.async_copy(src_hbm.at[pl.ds(e0, CHUNK)], srcva, semi0)
            ia1 = pltpu.async_copy(dst_hbm.at[pl.ds(e0, CHUNK)], dstva, semi0)
            ib0 = pltpu.async_copy(src_hbm.at[pl.ds(e0 + CHUNK, CHUNK)],
                                   srcvb, semi1)
            ib1 = pltpu.async_copy(dst_hbm.at[pl.ds(e0 + CHUNK, CHUNK)],
                                   dstvb, semi1)
            ia0.wait()
            ia1.wait()
            g0 = pltpu.async_copy(h_hbm.at[srcva], rows0, sem0)
            ib0.wait()
            ib1.wait()
            g1 = pltpu.async_copy(h_hbm.at[srcvb], rows1, sem1)
            g0.wait()
            pltpu.sync_copy(rows0, acc.at[dstva], add=True)
            g1.wait()
            pltpu.sync_copy(rows1, acc.at[dstvb], add=True)

        plsc.subcore_barrier()
        pltpu.sync_copy(acc.at[pl.ds(r0, ROWS_PER_TILE)],
                        out_hbm.at[c, pl.ds(r0, ROWS_PER_TILE)])

        @pl.when(s == NS - 1)
        def _():
            pltpu.sync_copy(acc.at[pl.ds(NS * ROWS_PER_TILE, ROWS_TAIL)],
                            out_hbm.at[c, pl.ds(NS * ROWS_PER_TILE, ROWS_TAIL)])

    return k(h, zeros, src2d, dst2d)


def _gin_mlp(p0, p1, W1, b1, W2, b2):
    BLK = 1000

    def body(p0_ref, p1_ref, w1_ref, b1_ref, w2_ref, b2_ref, o_ref):
        z = p0_ref[...] + p1_ref[...]
        h1 = jnp.dot(z, w1_ref[...], preferred_element_type=jnp.float32)
        h1 = jnp.maximum(h1 + b1_ref[...], 0.0)
        o_ref[...] = (jnp.dot(h1, w2_ref[...], preferred_element_type=jnp.float32)
                      + b2_ref[...])

    return pl.pallas_call(
        body,
        grid=(N // BLK,),
        in_specs=[
            pl.BlockSpec((BLK, D), lambda i: (i, 0)),
            pl.BlockSpec((BLK, D), lambda i: (i, 0)),
            pl.BlockSpec((D, D), lambda i: (0, 0)),
            pl.BlockSpec((1, D), lambda i: (0, 0)),
            pl.BlockSpec((D, D), lambda i: (0, 0)),
            pl.BlockSpec((1, D), lambda i: (0, 0)),
        ],
        out_specs=pl.BlockSpec((BLK, D), lambda i: (i, 0)),
        out_shape=jax.ShapeDtypeStruct((N, D), jnp.float32),
    )(p0, p1, W1, b1.reshape(1, D), W2, b2.reshape(1, D))


P_GATHER = 512
GPT = P_GATHER // NW


def _gather_sc(h, idx):
    @functools.partial(
        pl.kernel,
        mesh=_sc_mesh(),
        out_type=jax.ShapeDtypeStruct((P_GATHER, D), jnp.float32),
        scratch_types=[
            pltpu.VMEM((GPT,), jnp.int32),
            pltpu.VMEM((GPT, D), jnp.float32),
            pltpu.SemaphoreType.DMA,
        ],
    )
    def k(h_hbm, idx_hbm, out_hbm, idxv, rows, sem):
        c = lax.axis_index("c")
        s = lax.axis_index("s")
        w = s * NC + c
        pltpu.sync_copy(idx_hbm.at[pl.ds(w * GPT, GPT)], idxv)
        pltpu.async_copy(h_hbm.at[idxv], rows, sem).wait()
        pltpu.sync_copy(rows, out_hbm.at[pl.ds(w * GPT, GPT)])

    return k(h, idx)


def _scorer(xg, w_ts, w_ta, w_cs, w_ca, bp1, wp2t, bp2):
    def body(xg_ref, wts_ref, wta_ref, wcs_ref, wca_ref, bp1_ref, wp2t_ref,
             bp2_ref, o_ref):
        xu = xg_ref[0:1, :]
        xv = xg_ref[1:2, :]
        cu = xg_ref[8:136, :]
        cv = xg_ref[144:272, :]
        t = (jnp.dot(xu + xv, wts_ref[...], preferred_element_type=jnp.float32)
             + jnp.dot(jnp.abs(xu - xv), wta_ref[...],
                       preferred_element_type=jnp.float32))
        cmat = (jnp.dot(cu + cv, wcs_ref[...], preferred_element_type=jnp.float32)
                + jnp.dot(jnp.abs(cu - cv), wca_ref[...],
                          preferred_element_type=jnp.float32))
        act = jnp.maximum(cmat + t + bp1_ref[...], 0.0)
        o_ref[...] = (lax.dot_general(wp2t_ref[...], act,
                                      (((1,), (1,)), ((), ())),
                                      preferred_element_type=jnp.float32)
                      + bp2_ref[...])

    return pl.pallas_call(
        body,
        out_shape=jax.ShapeDtypeStruct((1, 128), jnp.float32),
    )(xg, w_ts, w_ta, w_cs, w_ca, bp1, wp2t, bp2)


def kernel(x, edge_index, edge_pairs, candidate_edges,
           W1_0, b1_0, W2_0, b2_0,
           W1_1, b1_1, W2_1, b2_1,
           Wp1, bp1, Wp2, bp2):
    src = edge_index[0]
    dst = edge_index[1]
    pad_src = jnp.zeros((EPAD - E,), jnp.int32)
    pad_dst = jnp.full((EPAD - E,), N, jnp.int32)
    src2d = jnp.concatenate([src, pad_src])
    dst2d = jnp.concatenate([dst, pad_dst])
    zeros = jnp.zeros((N, D), jnp.float32)

    p = _segsum_sc(x, zeros, src2d, dst2d)
    x1 = _gin_mlp(p[0], p[1], W1_0, b1_0, W2_0, b2_0)
    p = _segsum_sc(x1, zeros, src2d, dst2d)
    x2 = _gin_mlp(p[0], p[1], W1_1, b1_1, W2_1, b2_1)

    u = edge_pairs[:, 0]
    v = edge_pairs[:, 1]
    cu = candidate_edges[:, 0]
    cv = candidate_edges[:, 1]
    pad6 = jnp.zeros((6,), jnp.int32)
    pad8 = jnp.zeros((8,), jnp.int32)
    pad_tail = jnp.zeros((P_GATHER - 272,), jnp.int32)
    idx = jnp.concatenate([u, v, pad6, cu, pad8, cv, pad_tail])

    xg = _gather_sc(x2, idx)

    w_ts = Wp1[0:128]
    w_ta = Wp1[128:256]
    w_cs = Wp1[256:384]
    w_ca = Wp1[384:512]
    logits = _scorer(xg, w_ts, w_ta, w_cs, w_ca,
                     bp1.reshape(1, D), Wp2.T, bp2.reshape(1, 1))
    return logits


# speedup vs baseline: 2.7160x; 2.7160x over previous
import functools

import jax
import jax.numpy as jnp
from jax import lax
from jax.experimental import pallas as pl
from jax.experimental.pallas import tpu as pltpu
from jax.experimental.pallas import tpu_sc as plsc

N = 10000
D = 128
E = 320000

NC = 2
NS = 16
NW = NC * NS

CHUNK = 128
EDGES_PER_TILE = E // NW
NFULL = EDGES_PER_TILE // CHUNK
REM = EDGES_PER_TILE - NFULL * CHUNK
ROWS_PER_TILE = 624
ROWS_TAIL = N - NS * ROWS_PER_TILE

def _sc_mesh():
    return plsc.VectorSubcoreMesh(core_axis_name="c", subcore_axis_name="s")


def _segsum_sc(h, zeros, src2d, dst2d):
    @functools.partial(
        pl.kernel,
        mesh=_sc_mesh(),
        out_type=jax.ShapeDtypeStruct((NC, N, D), jnp.float32),
        scratch_types=[
            pltpu.VMEM_SHARED((N, D), jnp.float32),
            pltpu.VMEM((CHUNK,), jnp.int32),
            pltpu.VMEM((CHUNK,), jnp.int32),
            pltpu.VMEM((CHUNK,), jnp.int32),
            pltpu.VMEM((CHUNK,), jnp.int32),
            pltpu.VMEM((CHUNK, D), jnp.float32),
            pltpu.VMEM((CHUNK, D), jnp.float32),
            pltpu.VMEM((REM,), jnp.int32),
            pltpu.VMEM((REM,), jnp.int32),
            pltpu.VMEM((REM, D), jnp.float32),
            pltpu.SemaphoreType.DMA,
            pltpu.SemaphoreType.DMA,
            pltpu.SemaphoreType.DMA,
            pltpu.SemaphoreType.DMA,
        ],
    )
    def k(h_hbm, z_hbm, src_hbm, dst_hbm, out_hbm,
          acc, srcva, dstva, srcvb, dstvb, rows0, rows1,
          srcr, dstr, rowsr, semi0, semi1, sem0, sem1):
        c = lax.axis_index("c")
        s = lax.axis_index("s")
        w = s * NC + c
        r0 = s * ROWS_PER_TILE
        cbase = w * EDGES_PER_TILE

        @pl.when(c == 0)
        def _():
            pltpu.sync_copy(h_hbm.at[pl.ds(r0, ROWS_PER_TILE)],
                            acc.at[pl.ds(r0, ROWS_PER_TILE)])

        @pl.when(c != 0)
        def _():
            pltpu.sync_copy(z_hbm.at[pl.ds(r0, ROWS_PER_TILE)],
                            acc.at[pl.ds(r0, ROWS_PER_TILE)])

        @pl.when((c == 0) & (s == NS - 1))
        def _():
            pltpu.sync_copy(h_hbm.at[pl.ds(NS * ROWS_PER_TILE, ROWS_TAIL)],
                            acc.at[pl.ds(NS * ROWS_PER_TILE, ROWS_TAIL)])

        @pl.when((c != 0) & (s == NS - 1))
        def _():
            pltpu.sync_copy(z_hbm.at[pl.ds(NS * ROWS_PER_TILE, ROWS_TAIL)],
                            acc.at[pl.ds(NS * ROWS_PER_TILE, ROWS_TAIL)])

        plsc.subcore_barrier()

        @pl.loop(0, NFULL // 2)
        def _(jj):
            e0 = cbase + 2 * jj * CHUNK
            ia0 = pltpu.async_copy(src_hbm.at[pl.ds(e0, CHUNK)], srcva, semi0)
            ia1 = pltpu.async_copy(dst_hbm.at[pl.ds(e0, CHUNK)], dstva, semi0)
            ib0 = pltpu.async_copy(src_hbm.at[pl.ds(e0 + CHUNK, CHUNK)],
                                   srcvb, semi1)
            ib1 = pltpu.async_copy(dst_hbm.at[pl.ds(e0 + CHUNK, CHUNK)],
                                   dstvb, semi1)
            ia0.wait()
            ia1.wait()
            g0 = pltpu.async_copy(h_hbm.at[srcva], rows0, sem0)
            ib0.wait()
            ib1.wait()
            g1 = pltpu.async_copy(h_hbm.at[srcvb], rows1, sem1)
            g0.wait()
            pltpu.sync_copy(rows0, acc.at[dstva], add=True)
            g1.wait()
            pltpu.sync_copy(rows1, acc.at[dstvb], add=True)

        rbase = cbase + NFULL * CHUNK
        pltpu.sync_copy(src_hbm.at[pl.ds(rbase, REM)], srcr)
        pltpu.sync_copy(dst_hbm.at[pl.ds(rbase, REM)], dstr)
        pltpu.async_copy(h_hbm.at[srcr], rowsr, sem0).wait()
        pltpu.sync_copy(rowsr, acc.at[dstr], add=True)

        plsc.subcore_barrier()
        pltpu.sync_copy(acc.at[pl.ds(r0, ROWS_PER_TILE)],
                        out_hbm.at[c, pl.ds(r0, ROWS_PER_TILE)])

        @pl.when(s == NS - 1)
        def _():
            pltpu.sync_copy(acc.at[pl.ds(NS * ROWS_PER_TILE, ROWS_TAIL)],
                            out_hbm.at[c, pl.ds(NS * ROWS_PER_TILE, ROWS_TAIL)])

    return k(h, zeros, src2d, dst2d)


def _gin_mlp(p0, p1, W1, b1, W2, b2):
    BLK = 1000

    def body(p0_ref, p1_ref, w1_ref, b1_ref, w2_ref, b2_ref, o_ref):
        z = p0_ref[...] + p1_ref[...]
        h1 = jnp.dot(z, w1_ref[...], preferred_element_type=jnp.float32)
        h1 = jnp.maximum(h1 + b1_ref[...], 0.0)
        o_ref[...] = (jnp.dot(h1, w2_ref[...], preferred_element_type=jnp.float32)
                      + b2_ref[...])

    return pl.pallas_call(
        body,
        grid=(N // BLK,),
        in_specs=[
            pl.BlockSpec((BLK, D), lambda i: (i, 0)),
            pl.BlockSpec((BLK, D), lambda i: (i, 0)),
            pl.BlockSpec((D, D), lambda i: (0, 0)),
            pl.BlockSpec((1, D), lambda i: (0, 0)),
            pl.BlockSpec((D, D), lambda i: (0, 0)),
            pl.BlockSpec((1, D), lambda i: (0, 0)),
        ],
        out_specs=pl.BlockSpec((BLK, D), lambda i: (i, 0)),
        out_shape=jax.ShapeDtypeStruct((N, D), jnp.float32),
    )(p0, p1, W1, b1.reshape(1, D), W2, b2.reshape(1, D))


P_GATHER = 512
GPT = P_GATHER // NW


def _gather_sc(h, idx):
    @functools.partial(
        pl.kernel,
        mesh=_sc_mesh(),
        out_type=jax.ShapeDtypeStruct((P_GATHER, D), jnp.float32),
        scratch_types=[
            pltpu.VMEM((GPT,), jnp.int32),
            pltpu.VMEM((GPT, D), jnp.float32),
            pltpu.SemaphoreType.DMA,
        ],
    )
    def k(h_hbm, idx_hbm, out_hbm, idxv, rows, sem):
        c = lax.axis_index("c")
        s = lax.axis_index("s")
        w = s * NC + c
        pltpu.sync_copy(idx_hbm.at[pl.ds(w * GPT, GPT)], idxv)
        pltpu.async_copy(h_hbm.at[idxv], rows, sem).wait()
        pltpu.sync_copy(rows, out_hbm.at[pl.ds(w * GPT, GPT)])

    return k(h, idx)


def _scorer(xg, w_ts, w_ta, w_cs, w_ca, bp1, wp2t, bp2):
    def body(xg_ref, wts_ref, wta_ref, wcs_ref, wca_ref, bp1_ref, wp2t_ref,
             bp2_ref, o_ref):
        xu = xg_ref[0:1, :]
        xv = xg_ref[1:2, :]
        cu = xg_ref[8:136, :]
        cv = xg_ref[144:272, :]
        t = (jnp.dot(xu + xv, wts_ref[...], preferred_element_type=jnp.float32)
             + jnp.dot(jnp.abs(xu - xv), wta_ref[...],
                       preferred_element_type=jnp.float32))
        cmat = (jnp.dot(cu + cv, wcs_ref[...], preferred_element_type=jnp.float32)
                + jnp.dot(jnp.abs(cu - cv), wca_ref[...],
                          preferred_element_type=jnp.float32))
        act = jnp.maximum(cmat + t + bp1_ref[...], 0.0)
        o_ref[...] = (lax.dot_general(wp2t_ref[...], act,
                                      (((1,), (1,)), ((), ())),
                                      preferred_element_type=jnp.float32)
                      + bp2_ref[...])

    return pl.pallas_call(
        body,
        out_shape=jax.ShapeDtypeStruct((1, 128), jnp.float32),
    )(xg, w_ts, w_ta, w_cs, w_ca, bp1, wp2t, bp2)


def kernel(x, edge_index, edge_pairs, candidate_edges,
           W1_0, b1_0, W2_0, b2_0,
           W1_1, b1_1, W2_1, b2_1,
           Wp1, bp1, Wp2, bp2):
    src2d = edge_index[0]
    dst2d = edge_index[1]
    zeros = jnp.zeros((N, D), jnp.float32)

    p = _segsum_sc(x, zeros, src2d, dst2d)
    x1 = _gin_mlp(p[0], p[1], W1_0, b1_0, W2_0, b2_0)
    p = _segsum_sc(x1, zeros, src2d, dst2d)
    x2 = _gin_mlp(p[0], p[1], W1_1, b1_1, W2_1, b2_1)

    u = edge_pairs[:, 0]
    v = edge_pairs[:, 1]
    cu = candidate_edges[:, 0]
    cv = candidate_edges[:, 1]
    pad6 = jnp.zeros((6,), jnp.int32)
    pad8 = jnp.zeros((8,), jnp.int32)
    pad_tail = jnp.zeros((P_GATHER - 272,), jnp.int32)
    idx = jnp.concatenate([u, v, pad6, cu, pad8, cv, pad_tail])

    xg = _gather_sc(x2, idx)

    w_ts = Wp1[0:128]
    w_ta = Wp1[128:256]
    w_cs = Wp1[256:384]
    w_ca = Wp1[384:512]
    logits = _scorer(xg, w_ts, w_ta, w_cs, w_ca,
                     bp1.reshape(1, D), Wp2.T, bp2.reshape(1, 1))
    return logits


# speedup vs baseline: 2.9888x; 1.1004x over previous
import functools

import jax
import jax.numpy as jnp
from jax import lax
from jax.experimental import pallas as pl
from jax.experimental.pallas import tpu as pltpu
from jax.experimental.pallas import tpu_sc as plsc

N = 10000
D = 128
E = 320000

NC = 2
NS = 16
NW = NC * NS

CHUNK = 128
EDGES_PER_TILE = E // NW
NFULL = EDGES_PER_TILE // CHUNK
REM = EDGES_PER_TILE - NFULL * CHUNK
ROWS_PER_TILE = 624
ROWS_TAIL = N - NS * ROWS_PER_TILE

def _sc_mesh():
    return plsc.VectorSubcoreMesh(core_axis_name="c", subcore_axis_name="s")


def _segsum_sc(h, zeros, src2d, dst2d):
    @functools.partial(
        pl.kernel,
        mesh=_sc_mesh(),
        out_type=jax.ShapeDtypeStruct((NC, N, D), jnp.float32),
        scratch_types=[
            pltpu.VMEM_SHARED((N, D), jnp.float32),
            pltpu.VMEM((NFULL * CHUNK,), jnp.int32),
            pltpu.VMEM((CHUNK,), jnp.int32),
            pltpu.VMEM((CHUNK,), jnp.int32),
            pltpu.VMEM((CHUNK, D), jnp.float32),
            pltpu.VMEM((CHUNK, D), jnp.float32),
            pltpu.VMEM((REM,), jnp.int32),
            pltpu.VMEM((REM,), jnp.int32),
            pltpu.VMEM((REM, D), jnp.float32),
            pltpu.SemaphoreType.DMA,
            pltpu.SemaphoreType.DMA,
            pltpu.SemaphoreType.DMA,
            pltpu.SemaphoreType.DMA,
        ],
    )
    def k(h_hbm, z_hbm, src_hbm, dst_hbm, out_hbm,
          acc, src_flat, dstva, dstvb, rows0, rows1,
          srcr, dstr, rowsr, semi0, semi1, sem0, sem1):
        c = lax.axis_index("c")
        s = lax.axis_index("s")
        w = s * NC + c
        r0 = s * ROWS_PER_TILE
        cbase = w * EDGES_PER_TILE
        sp = pltpu.async_copy(src_hbm.at[pl.ds(cbase, NFULL * CHUNK)],
                              src_flat, semi0)

        @pl.when(c == 0)
        def _():
            pltpu.sync_copy(h_hbm.at[pl.ds(r0, ROWS_PER_TILE)],
                            acc.at[pl.ds(r0, ROWS_PER_TILE)])

        @pl.when(c != 0)
        def _():
            pltpu.sync_copy(z_hbm.at[pl.ds(r0, ROWS_PER_TILE)],
                            acc.at[pl.ds(r0, ROWS_PER_TILE)])

        @pl.when((c == 0) & (s == NS - 1))
        def _():
            pltpu.sync_copy(h_hbm.at[pl.ds(NS * ROWS_PER_TILE, ROWS_TAIL)],
                            acc.at[pl.ds(NS * ROWS_PER_TILE, ROWS_TAIL)])

        @pl.when((c != 0) & (s == NS - 1))
        def _():
            pltpu.sync_copy(z_hbm.at[pl.ds(NS * ROWS_PER_TILE, ROWS_TAIL)],
                            acc.at[pl.ds(NS * ROWS_PER_TILE, ROWS_TAIL)])

        sp.wait()
        plsc.subcore_barrier()

        @pl.loop(0, NFULL // 2)
        def _(jj):
            e0 = cbase + 2 * jj * CHUNK
            ia = pltpu.async_copy(dst_hbm.at[pl.ds(e0, CHUNK)], dstva, semi0)
            ib = pltpu.async_copy(dst_hbm.at[pl.ds(e0 + CHUNK, CHUNK)],
                                  dstvb, semi1)
            g0 = pltpu.async_copy(
                h_hbm.at[src_flat.at[pl.ds(2 * jj * CHUNK, CHUNK)]],
                rows0, sem0)
            g1 = pltpu.async_copy(
                h_hbm.at[src_flat.at[pl.ds((2 * jj + 1) * CHUNK, CHUNK)]],
                rows1, sem1)
            g0.wait()
            ia.wait()
            pltpu.sync_copy(rows0, acc.at[dstva], add=True)
            g1.wait()
            ib.wait()
            pltpu.sync_copy(rows1, acc.at[dstvb], add=True)

        rbase = cbase + NFULL * CHUNK
        pltpu.sync_copy(src_hbm.at[pl.ds(rbase, REM)], srcr)
        pltpu.sync_copy(dst_hbm.at[pl.ds(rbase, REM)], dstr)
        pltpu.async_copy(h_hbm.at[srcr], rowsr, sem0).wait()
        pltpu.sync_copy(rowsr, acc.at[dstr], add=True)

        plsc.subcore_barrier()
        pltpu.sync_copy(acc.at[pl.ds(r0, ROWS_PER_TILE)],
                        out_hbm.at[c, pl.ds(r0, ROWS_PER_TILE)])

        @pl.when(s == NS - 1)
        def _():
            pltpu.sync_copy(acc.at[pl.ds(NS * ROWS_PER_TILE, ROWS_TAIL)],
                            out_hbm.at[c, pl.ds(NS * ROWS_PER_TILE, ROWS_TAIL)])

    return k(h, zeros, src2d, dst2d)


def _gin_mlp(p0, p1, W1, b1, W2, b2):
    BLK = 1000

    def body(p0_ref, p1_ref, w1_ref, b1_ref, w2_ref, b2_ref, o_ref):
        z = p0_ref[...] + p1_ref[...]
        h1 = jnp.dot(z, w1_ref[...], preferred_element_type=jnp.float32)
        h1 = jnp.maximum(h1 + b1_ref[...], 0.0)
        o_ref[...] = (jnp.dot(h1, w2_ref[...], preferred_element_type=jnp.float32)
                      + b2_ref[...])

    return pl.pallas_call(
        body,
        grid=(N // BLK,),
        in_specs=[
            pl.BlockSpec((BLK, D), lambda i: (i, 0)),
            pl.BlockSpec((BLK, D), lambda i: (i, 0)),
            pl.BlockSpec((D, D), lambda i: (0, 0)),
            pl.BlockSpec((1, D), lambda i: (0, 0)),
            pl.BlockSpec((D, D), lambda i: (0, 0)),
            pl.BlockSpec((1, D), lambda i: (0, 0)),
        ],
        out_specs=pl.BlockSpec((BLK, D), lambda i: (i, 0)),
        out_shape=jax.ShapeDtypeStruct((N, D), jnp.float32),
    )(p0, p1, W1, b1.reshape(1, D), W2, b2.reshape(1, D))


P_GATHER = 512
GPT = P_GATHER // NW


def _gather_sc(h, idx):
    @functools.partial(
        pl.kernel,
        mesh=_sc_mesh(),
        out_type=jax.ShapeDtypeStruct((P_GATHER, D), jnp.float32),
        scratch_types=[
            pltpu.VMEM((GPT,), jnp.int32),
            pltpu.VMEM((GPT, D), jnp.float32),
            pltpu.SemaphoreType.DMA,
        ],
    )
    def k(h_hbm, idx_hbm, out_hbm, idxv, rows, sem):
        c = lax.axis_index("c")
        s = lax.axis_index("s")
        w = s * NC + c
        pltpu.sync_copy(idx_hbm.at[pl.ds(w * GPT, GPT)], idxv)
        pltpu.async_copy(h_hbm.at[idxv], rows, sem).wait()
        pltpu.sync_copy(rows, out_hbm.at[pl.ds(w * GPT, GPT)])

    return k(h, idx)


def _scorer(xg, w_ts, w_ta, w_cs, w_ca, bp1, wp2t, bp2):
    def body(xg_ref, wts_ref, wta_ref, wcs_ref, wca_ref, bp1_ref, wp2t_ref,
             bp2_ref, o_ref):
        xu = xg_ref[0:1, :]
        xv = xg_ref[1:2, :]
        cu = xg_ref[8:136, :]
        cv = xg_ref[144:272, :]
        t = (jnp.dot(xu + xv, wts_ref[...], preferred_element_type=jnp.float32)
             + jnp.dot(jnp.abs(xu - xv), wta_ref[...],
                       preferred_element_type=jnp.float32))
        cmat = (jnp.dot(cu + cv, wcs_ref[...], preferred_element_type=jnp.float32)
                + jnp.dot(jnp.abs(cu - cv), wca_ref[...],
                          preferred_element_type=jnp.float32))
        act = jnp.maximum(cmat + t + bp1_ref[...], 0.0)
        o_ref[...] = (lax.dot_general(wp2t_ref[...], act,
                                      (((1,), (1,)), ((), ())),
                                      preferred_element_type=jnp.float32)
                      + bp2_ref[...])

    return pl.pallas_call(
        body,
        out_shape=jax.ShapeDtypeStruct((1, 128), jnp.float32),
    )(xg, w_ts, w_ta, w_cs, w_ca, bp1, wp2t, bp2)


def kernel(x, edge_index, edge_pairs, candidate_edges,
           W1_0, b1_0, W2_0, b2_0,
           W1_1, b1_1, W2_1, b2_1,
           Wp1, bp1, Wp2, bp2):
    src2d = edge_index[0]
    dst2d = edge_index[1]
    zeros = jnp.zeros((N, D), jnp.float32)

    p = _segsum_sc(x, zeros, src2d, dst2d)
    x1 = _gin_mlp(p[0], p[1], W1_0, b1_0, W2_0, b2_0)
    p = _segsum_sc(x1, zeros, src2d, dst2d)
    x2 = _gin_mlp(p[0], p[1], W1_1, b1_1, W2_1, b2_1)

    u = edge_pairs[:, 0]
    v = edge_pairs[:, 1]
    cu = candidate_edges[:, 0]
    cv = candidate_edges[:, 1]
    pad6 = jnp.zeros((6,), jnp.int32)
    pad8 = jnp.zeros((8,), jnp.int32)
    pad_tail = jnp.zeros((P_GATHER - 272,), jnp.int32)
    idx = jnp.concatenate([u, v, pad6, cu, pad8, cv, pad_tail])

    xg = _gather_sc(x2, idx)

    w_ts = Wp1[0:128]
    w_ta = Wp1[128:256]
    w_cs = Wp1[256:384]
    w_ca = Wp1[384:512]
    logits = _scorer(xg, w_ts, w_ta, w_cs, w_ca,
                     bp1.reshape(1, D), Wp2.T, bp2.reshape(1, 1))
    return logits


# speedup vs baseline: 3.8397x; 1.2847x over previous
import functools

import jax
import jax.numpy as jnp
from jax import lax
from jax.experimental import pallas as pl
from jax.experimental.pallas import tpu as pltpu
from jax.experimental.pallas import tpu_sc as plsc

N = 10000
D = 128
E = 320000

NC = 2
NS = 16
NW = NC * NS

CHUNK = 128
EDGES_PER_TILE = E // NW
NFULL = EDGES_PER_TILE // CHUNK
REM = EDGES_PER_TILE - NFULL * CHUNK
ROWS_PER_TILE = 624
ROWS_TAIL = N - NS * ROWS_PER_TILE

def _sc_mesh():
    return plsc.VectorSubcoreMesh(core_axis_name="c", subcore_axis_name="s")


def _segsum_sc(h, zeros, src2d, dst2d):
    @functools.partial(
        pl.kernel,
        mesh=_sc_mesh(),
        out_type=jax.ShapeDtypeStruct((NC, N, D), jnp.float32),
        scratch_types=[
            pltpu.VMEM_SHARED((N, D), jnp.float32),
            pltpu.VMEM((NFULL * CHUNK,), jnp.int32),
            pltpu.VMEM((CHUNK,), jnp.int32),
            pltpu.VMEM((CHUNK,), jnp.int32),
            pltpu.VMEM((CHUNK, D), jnp.float32),
            pltpu.VMEM((CHUNK, D), jnp.float32),
            pltpu.VMEM((REM,), jnp.int32),
            pltpu.VMEM((REM,), jnp.int32),
            pltpu.VMEM((REM, D), jnp.float32),
            pltpu.SemaphoreType.DMA,
            pltpu.SemaphoreType.DMA,
            pltpu.SemaphoreType.DMA,
            pltpu.SemaphoreType.DMA,
        ],
    )
    def k(h_hbm, z_hbm, src_hbm, dst_hbm, out_hbm,
          acc, src_flat, dstva, dstvb, rows0, rows1,
          srcr, dstr, rowsr, semi0, semi1, sem0, sem1):
        c = lax.axis_index("c")
        s = lax.axis_index("s")
        w = s * NC + c
        r0 = s * ROWS_PER_TILE
        cbase = w * EDGES_PER_TILE
        sp = pltpu.async_copy(src_hbm.at[pl.ds(cbase, NFULL * CHUNK)],
                              src_flat, semi0)

        @pl.when(c == 0)
        def _():
            pltpu.sync_copy(h_hbm.at[pl.ds(r0, ROWS_PER_TILE)],
                            acc.at[pl.ds(r0, ROWS_PER_TILE)])

        @pl.when(c != 0)
        def _():
            pltpu.sync_copy(z_hbm.at[pl.ds(r0, ROWS_PER_TILE)],
                            acc.at[pl.ds(r0, ROWS_PER_TILE)])

        @pl.when((c == 0) & (s == NS - 1))
        def _():
            pltpu.sync_copy(h_hbm.at[pl.ds(NS * ROWS_PER_TILE, ROWS_TAIL)],
                            acc.at[pl.ds(NS * ROWS_PER_TILE, ROWS_TAIL)])

        @pl.when((c != 0) & (s == NS - 1))
        def _():
            pltpu.sync_copy(z_hbm.at[pl.ds(NS * ROWS_PER_TILE, ROWS_TAIL)],
                            acc.at[pl.ds(NS * ROWS_PER_TILE, ROWS_TAIL)])

        sp.wait()
        plsc.subcore_barrier()

        def _gather(chunk, rows, sem):
            return pltpu.make_async_copy(
                h_hbm.at[src_flat.at[pl.ds(chunk * CHUNK, CHUNK)]], rows, sem)

        def _dload(chunk, dstv, sem):
            return pltpu.make_async_copy(
                dst_hbm.at[pl.ds(cbase + chunk * CHUNK, CHUNK)], dstv, sem)

        _gather(0, rows0, sem0).start()
        _dload(0, dstva, semi0).start()

        @pl.loop(0, NFULL // 2)
        def _(jj):
            a = 2 * jj
            b = a + 1
            _gather(b, rows1, sem1).start()
            _dload(b, dstvb, semi1).start()
            _gather(a, rows0, sem0).wait()
            _dload(a, dstva, semi0).wait()
            pltpu.sync_copy(rows0, acc.at[dstva], add=True)

            @pl.when(jj < NFULL // 2 - 1)
            def _():
                _gather(a + 2, rows0, sem0).start()
                _dload(a + 2, dstva, semi0).start()

            _gather(b, rows1, sem1).wait()
            _dload(b, dstvb, semi1).wait()
            pltpu.sync_copy(rows1, acc.at[dstvb], add=True)

        rbase = cbase + NFULL * CHUNK
        pltpu.sync_copy(src_hbm.at[pl.ds(rbase, REM)], srcr)
        pltpu.sync_copy(dst_hbm.at[pl.ds(rbase, REM)], dstr)
        pltpu.async_copy(h_hbm.at[srcr], rowsr, sem0).wait()
        pltpu.sync_copy(rowsr, acc.at[dstr], add=True)

        plsc.subcore_barrier()
        pltpu.sync_copy(acc.at[pl.ds(r0, ROWS_PER_TILE)],
                        out_hbm.at[c, pl.ds(r0, ROWS_PER_TILE)])

        @pl.when(s == NS - 1)
        def _():
            pltpu.sync_copy(acc.at[pl.ds(NS * ROWS_PER_TILE, ROWS_TAIL)],
                            out_hbm.at[c, pl.ds(NS * ROWS_PER_TILE, ROWS_TAIL)])

    return k(h, zeros, src2d, dst2d)


def _gin_mlp(p0, p1, W1, b1, W2, b2):
    BLK = 1000

    def body(p0_ref, p1_ref, w1_ref, b1_ref, w2_ref, b2_ref, o_ref):
        z = p0_ref[...] + p1_ref[...]
        h1 = jnp.dot(z, w1_ref[...], preferred_element_type=jnp.float32)
        h1 = jnp.maximum(h1 + b1_ref[...], 0.0)
        o_ref[...] = (jnp.dot(h1, w2_ref[...], preferred_element_type=jnp.float32)
                      + b2_ref[...])

    return pl.pallas_call(
        body,
        grid=(N // BLK,),
        in_specs=[
            pl.BlockSpec((BLK, D), lambda i: (i, 0)),
            pl.BlockSpec((BLK, D), lambda i: (i, 0)),
            pl.BlockSpec((D, D), lambda i: (0, 0)),
            pl.BlockSpec((1, D), lambda i: (0, 0)),
            pl.BlockSpec((D, D), lambda i: (0, 0)),
            pl.BlockSpec((1, D), lambda i: (0, 0)),
        ],
        out_specs=pl.BlockSpec((BLK, D), lambda i: (i, 0)),
        out_shape=jax.ShapeDtypeStruct((N, D), jnp.float32),
    )(p0, p1, W1, b1.reshape(1, D), W2, b2.reshape(1, D))


P_GATHER = 512
GPT = P_GATHER // NW


def _gather_sc(h, idx):
    @functools.partial(
        pl.kernel,
        mesh=_sc_mesh(),
        out_type=jax.ShapeDtypeStruct((P_GATHER, D), jnp.float32),
        scratch_types=[
            pltpu.VMEM((GPT,), jnp.int32),
            pltpu.VMEM((GPT, D), jnp.float32),
            pltpu.SemaphoreType.DMA,
        ],
    )
    def k(h_hbm, idx_hbm, out_hbm, idxv, rows, sem):
        c = lax.axis_index("c")
        s = lax.axis_index("s")
        w = s * NC + c
        pltpu.sync_copy(idx_hbm.at[pl.ds(w * GPT, GPT)], idxv)
        pltpu.async_copy(h_hbm.at[idxv], rows, sem).wait()
        pltpu.sync_copy(rows, out_hbm.at[pl.ds(w * GPT, GPT)])

    return k(h, idx)


def _scorer(xg, w_ts, w_ta, w_cs, w_ca, bp1, wp2t, bp2):
    def body(xg_ref, wts_ref, wta_ref, wcs_ref, wca_ref, bp1_ref, wp2t_ref,
             bp2_ref, o_ref):
        xu = xg_ref[0:1, :]
        xv = xg_ref[1:2, :]
        cu = xg_ref[8:136, :]
        cv = xg_ref[144:272, :]
        t = (jnp.dot(xu + xv, wts_ref[...], preferred_element_type=jnp.float32)
             + jnp.dot(jnp.abs(xu - xv), wta_ref[...],
                       preferred_element_type=jnp.float32))
        cmat = (jnp.dot(cu + cv, wcs_ref[...], preferred_element_type=jnp.float32)
                + jnp.dot(jnp.abs(cu - cv), wca_ref[...],
                          preferred_element_type=jnp.float32))
        act = jnp.maximum(cmat + t + bp1_ref[...], 0.0)
        o_ref[...] = (lax.dot_general(wp2t_ref[...], act,
                                      (((1,), (1,)), ((), ())),
                                      preferred_element_type=jnp.float32)
                      + bp2_ref[...])

    return pl.pallas_call(
        body,
        out_shape=jax.ShapeDtypeStruct((1, 128), jnp.float32),
    )(xg, w_ts, w_ta, w_cs, w_ca, bp1, wp2t, bp2)


def kernel(x, edge_index, edge_pairs, candidate_edges,
           W1_0, b1_0, W2_0, b2_0,
           W1_1, b1_1, W2_1, b2_1,
           Wp1, bp1, Wp2, bp2):
    src2d = edge_index[0]
    dst2d = edge_index[1]
    zeros = jnp.zeros((N, D), jnp.float32)

    p = _segsum_sc(x, zeros, src2d, dst2d)
    x1 = _gin_mlp(p[0], p[1], W1_0, b1_0, W2_0, b2_0)
    p = _segsum_sc(x1, zeros, src2d, dst2d)
    x2 = _gin_mlp(p[0], p[1], W1_1, b1_1, W2_1, b2_1)

    u = edge_pairs[:, 0]
    v = edge_pairs[:, 1]
    cu = candidate_edges[:, 0]
    cv = candidate_edges[:, 1]
    pad6 = jnp.zeros((6,), jnp.int32)
    pad8 = jnp.zeros((8,), jnp.int32)
    pad_tail = jnp.zeros((P_GATHER - 272,), jnp.int32)
    idx = jnp.concatenate([u, v, pad6, cu, pad8, cv, pad_tail])

    xg = _gather_sc(x2, idx)

    w_ts = Wp1[0:128]
    w_ta = Wp1[128:256]
    w_cs = Wp1[256:384]
    w_ca = Wp1[384:512]
    logits = _scorer(xg, w_ts, w_ta, w_cs, w_ca,
                     bp1.reshape(1, D), Wp2.T, bp2.reshape(1, 1))
    return logits


# speedup vs baseline: 4.0673x; 1.0593x over previous
import functools

import jax
import jax.numpy as jnp
from jax import lax
from jax.experimental import pallas as pl
from jax.experimental.pallas import tpu as pltpu
from jax.experimental.pallas import tpu_sc as plsc

N = 10000
D = 128
E = 320000

NC = 2
NS = 16
NW = NC * NS

CHUNK = 128
EDGES_PER_TILE = E // NW
NFULL = EDGES_PER_TILE // CHUNK
REM = EDGES_PER_TILE - NFULL * CHUNK
ROWS_PER_TILE = 624
ROWS_TAIL = N - NS * ROWS_PER_TILE

def _sc_mesh():
    return plsc.VectorSubcoreMesh(core_axis_name="c", subcore_axis_name="s")


def _segsum_sc(h, zeros, src2d, dst2d):
    @functools.partial(
        pl.kernel,
        mesh=_sc_mesh(),
        out_type=jax.ShapeDtypeStruct((NC, N, D), jnp.float32),
        scratch_types=[
            pltpu.VMEM_SHARED((N, D), jnp.float32),
            pltpu.VMEM((NFULL * CHUNK,), jnp.int32),
            pltpu.VMEM((CHUNK,), jnp.int32),
            pltpu.VMEM((CHUNK,), jnp.int32),
            pltpu.VMEM((CHUNK, D), jnp.float32),
            pltpu.VMEM((CHUNK, D), jnp.float32),
            pltpu.VMEM((REM,), jnp.int32),
            pltpu.VMEM((REM,), jnp.int32),
            pltpu.VMEM((REM, D), jnp.float32),
            pltpu.SemaphoreType.DMA,
            pltpu.SemaphoreType.DMA,
            pltpu.SemaphoreType.DMA,
            pltpu.SemaphoreType.DMA,
        ],
    )
    def k(h_hbm, z_hbm, src_hbm, dst_hbm, out_hbm,
          acc, src_flat, dstva, dstvb, rows0, rows1,
          srcr, dstr, rowsr, semi0, semi1, sem0, sem1):
        c = lax.axis_index("c")
        s = lax.axis_index("s")
        w = s * NC + c
        r0 = s * ROWS_PER_TILE
        cbase = w * EDGES_PER_TILE
        sp = pltpu.async_copy(src_hbm.at[pl.ds(cbase, NFULL * CHUNK)],
                              src_flat, semi0)

        @pl.when(c == 0)
        def _():
            pltpu.sync_copy(h_hbm.at[pl.ds(r0, ROWS_PER_TILE)],
                            acc.at[pl.ds(r0, ROWS_PER_TILE)])

        @pl.when(c != 0)
        def _():
            pltpu.sync_copy(z_hbm.at[pl.ds(r0, ROWS_PER_TILE)],
                            acc.at[pl.ds(r0, ROWS_PER_TILE)])

        @pl.when((c == 0) & (s == NS - 1))
        def _():
            pltpu.sync_copy(h_hbm.at[pl.ds(NS * ROWS_PER_TILE, ROWS_TAIL)],
                            acc.at[pl.ds(NS * ROWS_PER_TILE, ROWS_TAIL)])

        @pl.when((c != 0) & (s == NS - 1))
        def _():
            pltpu.sync_copy(z_hbm.at[pl.ds(NS * ROWS_PER_TILE, ROWS_TAIL)],
                            acc.at[pl.ds(NS * ROWS_PER_TILE, ROWS_TAIL)])

        sp.wait()
        plsc.subcore_barrier()

        def _gather(chunk, rows, sem):
            return pltpu.make_async_copy(
                h_hbm.at[src_flat.at[pl.ds(chunk * CHUNK, CHUNK)]], rows, sem)

        def _dload(chunk, dstv, sem):
            return pltpu.make_async_copy(
                dst_hbm.at[pl.ds(cbase + chunk * CHUNK, CHUNK)], dstv, sem)

        _gather(0, rows0, sem0).start()
        _dload(0, dstva, semi0).start()

        @pl.loop(0, NFULL // 2)
        def _(jj):
            a = 2 * jj
            b = a + 1
            _gather(b, rows1, sem1).start()
            _dload(b, dstvb, semi1).start()
            _gather(a, rows0, sem0).wait()
            _dload(a, dstva, semi0).wait()
            pltpu.sync_copy(rows0, acc.at[dstva], add=True)

            @pl.when(jj < NFULL // 2 - 1)
            def _():
                _gather(a + 2, rows0, sem0).start()
                _dload(a + 2, dstva, semi0).start()

            _gather(b, rows1, sem1).wait()
            _dload(b, dstvb, semi1).wait()
            pltpu.sync_copy(rows1, acc.at[dstvb], add=True)

        rbase = cbase + NFULL * CHUNK
        pltpu.sync_copy(src_hbm.at[pl.ds(rbase, REM)], srcr)
        pltpu.sync_copy(dst_hbm.at[pl.ds(rbase, REM)], dstr)
        pltpu.async_copy(h_hbm.at[srcr], rowsr, sem0).wait()
        pltpu.sync_copy(rowsr, acc.at[dstr], add=True)

        plsc.subcore_barrier()
        pltpu.sync_copy(acc.at[pl.ds(r0, ROWS_PER_TILE)],
                        out_hbm.at[c, pl.ds(r0, ROWS_PER_TILE)])

        @pl.when(s == NS - 1)
        def _():
            pltpu.sync_copy(acc.at[pl.ds(NS * ROWS_PER_TILE, ROWS_TAIL)],
                            out_hbm.at[c, pl.ds(NS * ROWS_PER_TILE, ROWS_TAIL)])

    return k(h, zeros, src2d, dst2d)


def _gin_mlp(p0, p1, W1, b1, W2, b2):
    BLK = 1000

    def body(p0_ref, p1_ref, w1_ref, b1_ref, w2_ref, b2_ref, o_ref):
        z = p0_ref[...] + p1_ref[...]
        h1 = jnp.dot(z, w1_ref[...], preferred_element_type=jnp.float32)
        h1 = jnp.maximum(h1 + b1_ref[...], 0.0)
        o_ref[...] = (jnp.dot(h1, w2_ref[...], preferred_element_type=jnp.float32)
                      + b2_ref[...])

    return pl.pallas_call(
        body,
        grid=(N // BLK,),
        in_specs=[
            pl.BlockSpec((BLK, D), lambda i: (i, 0)),
            pl.BlockSpec((BLK, D), lambda i: (i, 0)),
            pl.BlockSpec((D, D), lambda i: (0, 0)),
            pl.BlockSpec((1, D), lambda i: (0, 0)),
            pl.BlockSpec((D, D), lambda i: (0, 0)),
            pl.BlockSpec((1, D), lambda i: (0, 0)),
        ],
        out_specs=pl.BlockSpec((BLK, D), lambda i: (i, 0)),
        out_shape=jax.ShapeDtypeStruct((N, D), jnp.float32),
    )(p0, p1, W1, b1.reshape(1, D), W2, b2.reshape(1, D))


P_GATHER = 512
PG_ALL = 2 * P_GATHER
GPT = PG_ALL // NW


def _gather_sc(h, idx):
    @functools.partial(
        pl.kernel,
        mesh=_sc_mesh(),
        out_type=jax.ShapeDtypeStruct((PG_ALL, D), jnp.float32),
        scratch_types=[
            pltpu.VMEM((GPT,), jnp.int32),
            pltpu.VMEM((GPT, D), jnp.float32),
            pltpu.SemaphoreType.DMA,
        ],
    )
    def k(h_hbm, idx_hbm, out_hbm, idxv, rows, sem):
        c = lax.axis_index("c")
        s = lax.axis_index("s")
        w = s * NC + c
        pltpu.sync_copy(idx_hbm.at[pl.ds(w * GPT, GPT)], idxv)
        pltpu.async_copy(h_hbm.at[idxv], rows, sem).wait()
        pltpu.sync_copy(rows, out_hbm.at[pl.ds(w * GPT, GPT)])

    return k(h, idx)


def _scorer(xg, W1, b1, W2, b2, w_ts, w_ta, w_cs, w_ca, bp1, wp2t, bp2):
    def body(xg_ref, w1_ref, b1_ref, w2_ref, b2_ref,
             wts_ref, wta_ref, wcs_ref, wca_ref, bp1_ref, wp2t_ref,
             bp2_ref, o_ref):
        z = xg_ref[0:P_GATHER, :] + xg_ref[P_GATHER:PG_ALL, :]
        h1 = jnp.dot(z, w1_ref[...], preferred_element_type=jnp.float32)
        h1 = jnp.maximum(h1 + b1_ref[...], 0.0)
        x2r = (jnp.dot(h1, w2_ref[...], preferred_element_type=jnp.float32)
               + b2_ref[...])
        xu = x2r[0:1, :]
        xv = x2r[1:2, :]
        cu = x2r[8:136, :]
        cv = x2r[144:272, :]
        t = (jnp.dot(xu + xv, wts_ref[...], preferred_element_type=jnp.float32)
             + jnp.dot(jnp.abs(xu - xv), wta_ref[...],
                       preferred_element_type=jnp.float32))
        cmat = (jnp.dot(cu + cv, wcs_ref[...], preferred_element_type=jnp.float32)
                + jnp.dot(jnp.abs(cu - cv), wca_ref[...],
                          preferred_element_type=jnp.float32))
        act = jnp.maximum(cmat + t + bp1_ref[...], 0.0)
        o_ref[...] = (lax.dot_general(wp2t_ref[...], act,
                                      (((1,), (1,)), ((), ())),
                                      preferred_element_type=jnp.float32)
                      + bp2_ref[...])

    return pl.pallas_call(
        body,
        out_shape=jax.ShapeDtypeStruct((1, 128), jnp.float32),
    )(xg, W1, b1.reshape(1, D), W2, b2.reshape(1, D),
      w_ts, w_ta, w_cs, w_ca, bp1, wp2t, bp2)


def kernel(x, edge_index, edge_pairs, candidate_edges,
           W1_0, b1_0, W2_0, b2_0,
           W1_1, b1_1, W2_1, b2_1,
           Wp1, bp1, Wp2, bp2):
    src2d = edge_index[0]
    dst2d = edge_index[1]
    zeros = jnp.zeros((N, D), jnp.float32)

    p = _segsum_sc(x, zeros, src2d, dst2d)
    x1 = _gin_mlp(p[0], p[1], W1_0, b1_0, W2_0, b2_0)
    p = _segsum_sc(x1, zeros, src2d, dst2d)

    u = edge_pairs[:, 0]
    v = edge_pairs[:, 1]
    cu = candidate_edges[:, 0]
    cv = candidate_edges[:, 1]
    pad6 = jnp.zeros((6,), jnp.int32)
    pad8 = jnp.zeros((8,), jnp.int32)
    pad_tail = jnp.zeros((P_GATHER - 272,), jnp.int32)
    idx = jnp.concatenate([u, v, pad6, cu, pad8, cv, pad_tail])
    idx2 = jnp.concatenate([idx, idx + N])

    xg = _gather_sc(p.reshape(2 * N, D), idx2)

    w_ts = Wp1[0:128]
    w_ta = Wp1[128:256]
    w_cs = Wp1[256:384]
    w_ca = Wp1[384:512]
    logits = _scorer(xg, W1_1, b1_1, W2_1, b2_1,
                     w_ts, w_ta, w_cs, w_ca,
                     bp1.reshape(1, D), Wp2.T, bp2.reshape(1, 1))
    return logits
